# Initial kernel scaffold; baseline (speedup 1.0000x reference)
import jax
import jax.numpy as jnp
from jax.experimental import pallas as pl


def kernel(X, edge_idx, C):
    raise NotImplementedError("write your pallas kernel here")


# speedup vs baseline: 2.8922x; 2.8922x over previous
import functools

import numpy as np
import jax
import jax.numpy as jnp
from jax import lax
from jax.experimental import pallas as pl

D_MODEL = 128
PERIOD_RANGE = (1.0, 1000.0)
NUM_FREQ = D_MODEL // 2

_log_bounds = np.log10(np.array(PERIOD_RANGE, dtype=np.float64))
_p = np.logspace(_log_bounds[0], _log_bounds[1], NUM_FREQ, base=10.0)
_w = (2.0 * np.pi / _p).astype(np.float32)
_W128 = np.concatenate([_w, _w]).reshape(1, D_MODEL).astype(np.float32)
_PH128 = np.concatenate(
    [np.zeros(NUM_FREQ), np.full(NUM_FREQ, -0.5 * np.pi)]
).reshape(1, D_MODEL).astype(np.float32)


def _encode_body(e_ref, c_ref, w_ref, ph_ref, o_ref, *, rows_per_blk, k, n_nodes):
    i = pl.program_id(0)
    e = e_ref[...]
    c = c_ref[...]
    row = i * rows_per_blk + lax.broadcasted_iota(
        jnp.int32, (rows_per_blk, 1), 0
    )
    n = row // k
    ve = jnp.zeros((rows_per_blk, 1), jnp.int32)
    vn = jnp.zeros((rows_per_blk, 1), jnp.int32)
    for v in range(1, 8):
        bv = jnp.sum((c < v).astype(jnp.int32))
        ve += (e >= bv).astype(jnp.int32)
        vn += (n >= bv).astype(jnp.int32)
    mask = ve == vn
    d = (e - n).astype(jnp.float32)
    ang = d * w_ref[...] + ph_ref[...]
    o_ref[...] = jnp.where(mask, jnp.cos(ang), 0.0)


@functools.partial(jax.jit, static_argnames=())
def kernel(X, edge_idx, C):
    del X
    B, N, K = edge_idx.shape
    e2 = edge_idx.reshape(N * K, 1).astype(jnp.int32)
    c2 = C.reshape(1, N).astype(jnp.int32)
    rows = N * K
    rows_per_blk = 6400
    grid = rows // rows_per_blk
    out = pl.pallas_call(
        functools.partial(
            _encode_body, rows_per_blk=rows_per_blk, k=K, n_nodes=N
        ),
        grid=(grid,),
        in_specs=[
            pl.BlockSpec((rows_per_blk, 1), lambda i: (i, 0)),
            pl.BlockSpec((1, N), lambda i: (0, 0)),
            pl.BlockSpec((1, D_MODEL), lambda i: (0, 0)),
            pl.BlockSpec((1, D_MODEL), lambda i: (0, 0)),
        ],
        out_specs=pl.BlockSpec((rows_per_blk, D_MODEL), lambda i: (i, 0)),
        out_shape=jax.ShapeDtypeStruct((rows, D_MODEL), jnp.float32),
    )(e2, c2, jnp.asarray(_W128), jnp.asarray(_PH128))
    return out.reshape(B, N, K, D_MODEL)
